# baseline (speedup 1.0000x reference)
import jax
import jax.numpy as jnp
import numpy as np
from jax.experimental import pallas as pl

B = 8
N = 1024
K = 16
C_IN = 64
C_OUT = 128
C_LIFT = 64
TILE = 256
Q = N // 4
BIG = 1e9
HUGE = 3e9
INT_BIG = 2 ** 30
HI = jax.lax.Precision.HIGHEST


def _ce(va, ia, vb, ib):
    pred = (va > vb) | ((va == vb) & (ia > ib))
    return (jnp.where(pred, vb, va), jnp.where(pred, ib, ia),
            jnp.where(pred, va, vb), jnp.where(pred, ia, ib))


def _xconv_kernel(q_ref, p_ref, pT_ref, feat_ref, W1_ref, b1_ref, W2_ref,
                  b2_ref, Wl_ref, bl_ref, Wf_ref, bf_ref, out_ref):
    q = q_ref[0]
    p_full = p_ref[0]
    pT = pT_ref[0]
    feat = feat_ref[0]

    p0 = pT[0:1, :]
    p1 = pT[1:2, :]
    p2 = pT[2:3, :]
    valid_p = (p0 != 0.0) | (p1 != 0.0) | (p2 != 0.0)
    valid_col = jnp.any(p_full != 0.0, axis=1, keepdims=True)
    valid_q = jnp.any(q != 0.0, axis=1, keepdims=True)

    d0 = q[:, 0:1] - p0
    d1 = q[:, 1:2] - p1
    d2 = q[:, 2:3] - p2
    pd = d0 * d0 + d1 * d1 + d2 * d2
    pd = jnp.where(valid_q & valid_p, pd, BIG)

    lifted = jax.nn.relu(jnp.dot(feat, Wl_ref[...],
                                 preferred_element_type=jnp.float32,
                                 precision=HI)
                         + bl_ref[...])
    lifted = jnp.where(valid_col, lifted, 0.0)

    payload = jnp.concatenate(
        [p_full, valid_col.astype(jnp.float32), lifted], axis=1)

    base = jax.lax.broadcasted_iota(jnp.int32, (TILE, Q), 1)
    v0, i0 = pd[:, 0:Q], base
    v1, i1 = pd[:, Q:2 * Q], base + Q
    v2, i2 = pd[:, 2 * Q:3 * Q], base + 2 * Q
    v3, i3 = pd[:, 3 * Q:4 * Q], base + 3 * Q
    v0, i0, v1, i1 = _ce(v0, i0, v1, i1)
    v2, i2, v3, i3 = _ce(v2, i2, v3, i3)
    v0, i0, v2, i2 = _ce(v0, i0, v2, i2)
    v1, i1, v3, i3 = _ce(v1, i1, v3, i3)
    v1, i1, v2, i2 = _ce(v1, i1, v2, i2)

    iota_full = jax.lax.broadcasted_iota(jnp.int32, (TILE, N), 1)

    h_parts = []
    L_parts = []
    for _ in range(K):
        m = jnp.min(v0, axis=1, keepdims=True)
        cand = jnp.where(v0 <= m, i0, INT_BIG)
        argi = jnp.min(cand, axis=1, keepdims=True)
        pool_oh = i0 == argi
        v0 = jnp.where(pool_oh, v1, v0)
        i0 = jnp.where(pool_oh, i1, i0)
        v1 = jnp.where(pool_oh, v2, v1)
        i1 = jnp.where(pool_oh, i2, i1)
        v2 = jnp.where(pool_oh, v3, v2)
        i2 = jnp.where(pool_oh, i3, i2)
        v3 = jnp.where(pool_oh, HUGE, v3)
        i3 = jnp.where(pool_oh, INT_BIG, i3)

        ohf = jnp.where(iota_full == argi, 1.0, 0.0)
        g = jnp.dot(ohf, payload, preferred_element_type=jnp.float32,
                    precision=HI)
        nbr_valid = g[:, 3:4] > 0.5
        rel = jnp.where(nbr_valid, g[:, 0:3] - q, 0.0)
        h = jax.nn.relu(jnp.dot(rel, W1_ref[...],
                                preferred_element_type=jnp.float32,
                                precision=HI)
                        + b1_ref[...])
        h_parts.append(h)
        L_parts.append(g[:, 4:4 + C_LIFT])

    h_flat = jnp.concatenate(h_parts, axis=1)
    Xp = jnp.dot(h_flat, W2_ref[...],
                 preferred_element_type=jnp.float32,
                 precision=HI) + b2_ref[...]

    lane_e = jax.lax.broadcasted_iota(jnp.int32, (K, N), 1)
    row_e = jax.lax.broadcasted_iota(jnp.int32, (K, N), 0)
    E = (lane_e // C_LIFT == row_e).astype(jnp.float32)
    lane_f = jax.lax.broadcasted_iota(jnp.int32, (C_LIFT, N), 1)
    row_f = jax.lax.broadcasted_iota(jnp.int32, (C_LIFT, N), 0)
    F = (lane_f % C_LIFT == row_f).astype(jnp.float32)

    T = None
    for j in range(K):
        Xrep = jnp.dot(Xp[:, j * K:(j + 1) * K], E,
                       preferred_element_type=jnp.float32, precision=HI)
        Ltile = jnp.dot(L_parts[j], F,
                        preferred_element_type=jnp.float32, precision=HI)
        T = Xrep * Ltile if T is None else T + Xrep * Ltile

    final = jax.nn.relu(jnp.dot(T, Wf_ref[...],
                                preferred_element_type=jnp.float32,
                                precision=HI)
                        + bf_ref[...])
    out_ref[0] = jnp.where(valid_q, final, 0.0)


@jax.jit
def kernel(points_xyz, features, W1, b1, W2, b2, Wl, bl, Wf, bf):
    pT = jnp.transpose(points_xyz, (0, 2, 1))
    perm = np.arange(K * K).reshape(K, K).T.reshape(-1)
    W2p = W2[:, perm]
    b2p = b2[perm]
    grid = (B, N // TILE)
    out = pl.pallas_call(
        _xconv_kernel,
        grid=grid,
        in_specs=[
            pl.BlockSpec((1, TILE, 3), lambda b, t: (b, t, 0)),
            pl.BlockSpec((1, N, 3), lambda b, t: (b, 0, 0)),
            pl.BlockSpec((1, 3, N), lambda b, t: (b, 0, 0)),
            pl.BlockSpec((1, N, C_IN), lambda b, t: (b, 0, 0)),
            pl.BlockSpec(W1.shape, lambda b, t: (0, 0)),
            pl.BlockSpec((1, K * 2), lambda b, t: (0, 0)),
            pl.BlockSpec(W2.shape, lambda b, t: (0, 0)),
            pl.BlockSpec((1, K * K), lambda b, t: (0, 0)),
            pl.BlockSpec(Wl.shape, lambda b, t: (0, 0)),
            pl.BlockSpec((1, C_LIFT), lambda b, t: (0, 0)),
            pl.BlockSpec(Wf.shape, lambda b, t: (0, 0)),
            pl.BlockSpec((1, C_OUT), lambda b, t: (0, 0)),
        ],
        out_specs=pl.BlockSpec((1, TILE, C_OUT), lambda b, t: (b, t, 0)),
        out_shape=jax.ShapeDtypeStruct((B, N, C_OUT), jnp.float32),
    )(points_xyz, points_xyz, pT, features, W1, b1.reshape(1, -1), W2p,
      b2p.reshape(1, -1), Wl, bl.reshape(1, -1), Wf, bf.reshape(1, -1))
    return out


# speedup vs baseline: 2.9992x; 2.9992x over previous
import jax
import jax.numpy as jnp
import numpy as np
from jax.experimental import pallas as pl
from jax.experimental.pallas import tpu as pltpu

B = 8
N = 1024
K = 16
C_IN = 64
C_OUT = 128
C_LIFT = 64
TILE = 256
Q = N // 4
BIG = 1e9
HUGE = 3e9
IDX_BIG = 1e9
HI = jax.lax.Precision.HIGHEST


def _ce(va, ia, vb, ib):
    pred = (va > vb) | ((va == vb) & (ia > ib))
    return (jnp.where(pred, vb, va), jnp.where(pred, ib, ia),
            jnp.where(pred, va, vb), jnp.where(pred, ia, ib))


def _xconv_kernel(q_ref, p_ref, pT_ref, feat_ref, W1_ref, b1_ref, W2_ref,
                  b2_ref, Wl_ref, bl_ref, Wf_ref, bf_ref, out_ref):
    q = q_ref[0]
    p_full = p_ref[0]
    pT = pT_ref[0]
    feat = feat_ref[0]

    p0 = pT[0:1, :]
    p1 = pT[1:2, :]
    p2 = pT[2:3, :]
    valid_p = (p0 != 0.0) | (p1 != 0.0) | (p2 != 0.0)
    valid_col = jnp.any(p_full != 0.0, axis=1, keepdims=True)
    valid_q = jnp.any(q != 0.0, axis=1, keepdims=True)

    d0 = q[:, 0:1] - p0
    d1 = q[:, 1:2] - p1
    d2 = q[:, 2:3] - p2
    pd = d0 * d0 + d1 * d1 + d2 * d2
    pd = jnp.where(valid_q & valid_p, pd, BIG)

    lifted = jax.nn.relu(jnp.dot(feat, Wl_ref[...],
                                 preferred_element_type=jnp.float32)
                         + bl_ref[...])
    lifted = jnp.where(valid_col, lifted, 0.0)

    payload = jnp.concatenate(
        [p_full, valid_col.astype(jnp.float32), lifted], axis=1)

    base = jax.lax.broadcasted_iota(jnp.int32, (TILE, Q), 1).astype(jnp.float32)
    v0, i0 = pd[:, 0:Q], base
    v1, i1 = pd[:, Q:2 * Q], base + float(Q)
    v2, i2 = pd[:, 2 * Q:3 * Q], base + float(2 * Q)
    v3, i3 = pd[:, 3 * Q:4 * Q], base + float(3 * Q)
    v0, i0, v1, i1 = _ce(v0, i0, v1, i1)
    v2, i2, v3, i3 = _ce(v2, i2, v3, i3)
    v0, i0, v2, i2 = _ce(v0, i0, v2, i2)
    v1, i1, v3, i3 = _ce(v1, i1, v3, i3)
    v1, i1, v2, i2 = _ce(v1, i1, v2, i2)

    iota_full = jax.lax.broadcasted_iota(
        jnp.int32, (TILE, N), 1).astype(jnp.float32)

    h_parts = []
    L_parts = []
    for _ in range(K):
        m = jnp.min(v0, axis=1, keepdims=True)
        cand = jnp.where(v0 <= m, i0, IDX_BIG)
        argi = jnp.min(cand, axis=1, keepdims=True)
        pool_oh = i0 == argi
        v0 = jnp.where(pool_oh, v1, v0)
        i0 = jnp.where(pool_oh, i1, i0)
        v1 = jnp.where(pool_oh, v2, v1)
        i1 = jnp.where(pool_oh, i2, i1)
        v2 = jnp.where(pool_oh, v3, v2)
        i2 = jnp.where(pool_oh, i3, i2)
        v3 = jnp.where(pool_oh, HUGE, v3)
        i3 = jnp.where(pool_oh, IDX_BIG, i3)

        ohf = jnp.where(iota_full == argi, 1.0, 0.0)
        g = jnp.dot(ohf, payload, preferred_element_type=jnp.float32)
        nbr_valid = g[:, 3:4] > 0.5
        rel = jnp.where(nbr_valid, g[:, 0:3] - q, 0.0)
        h = jax.nn.relu(jnp.dot(rel, W1_ref[...],
                                preferred_element_type=jnp.float32)
                        + b1_ref[...])
        h_parts.append(h)
        L_parts.append(g[:, 4:4 + C_LIFT])

    h_flat = jnp.concatenate(h_parts, axis=1)
    Xp = jnp.dot(h_flat, W2_ref[...],
                 preferred_element_type=jnp.float32) + b2_ref[...]

    lane_e = jax.lax.broadcasted_iota(jnp.int32, (K, N), 1)
    row_e = jax.lax.broadcasted_iota(jnp.int32, (K, N), 0)
    E = (lane_e // C_LIFT == row_e).astype(jnp.float32)

    T = None
    for j in range(K):
        Xrep = jnp.dot(Xp[:, j * K:(j + 1) * K], E,
                       preferred_element_type=jnp.float32, precision=HI)
        Ltile = pltpu.repeat(L_parts[j], K, axis=1)
        T = Xrep * Ltile if T is None else T + Xrep * Ltile

    final = jax.nn.relu(jnp.dot(T, Wf_ref[...],
                                preferred_element_type=jnp.float32)
                        + bf_ref[...])
    out_ref[0] = jnp.where(valid_q, final, 0.0)


@jax.jit
def kernel(points_xyz, features, W1, b1, W2, b2, Wl, bl, Wf, bf):
    pT = jnp.transpose(points_xyz, (0, 2, 1))
    perm = np.arange(K * K).reshape(K, K).T.reshape(-1)
    W2p = W2[:, perm]
    b2p = b2[perm]
    grid = (B, N // TILE)
    out = pl.pallas_call(
        _xconv_kernel,
        grid=grid,
        in_specs=[
            pl.BlockSpec((1, TILE, 3), lambda b, t: (b, t, 0)),
            pl.BlockSpec((1, N, 3), lambda b, t: (b, 0, 0)),
            pl.BlockSpec((1, 3, N), lambda b, t: (b, 0, 0)),
            pl.BlockSpec((1, N, C_IN), lambda b, t: (b, 0, 0)),
            pl.BlockSpec(W1.shape, lambda b, t: (0, 0)),
            pl.BlockSpec((1, K * 2), lambda b, t: (0, 0)),
            pl.BlockSpec(W2.shape, lambda b, t: (0, 0)),
            pl.BlockSpec((1, K * K), lambda b, t: (0, 0)),
            pl.BlockSpec(Wl.shape, lambda b, t: (0, 0)),
            pl.BlockSpec((1, C_LIFT), lambda b, t: (0, 0)),
            pl.BlockSpec(Wf.shape, lambda b, t: (0, 0)),
            pl.BlockSpec((1, C_OUT), lambda b, t: (0, 0)),
        ],
        out_specs=pl.BlockSpec((1, TILE, C_OUT), lambda b, t: (b, t, 0)),
        out_shape=jax.ShapeDtypeStruct((B, N, C_OUT), jnp.float32),
    )(points_xyz, points_xyz, pT, features, W1, b1.reshape(1, -1), W2p,
      b2p.reshape(1, -1), Wl, bl.reshape(1, -1), Wf, bf.reshape(1, -1))
    return out


# speedup vs baseline: 5.2355x; 1.7456x over previous
import jax
import jax.numpy as jnp
import numpy as np
from jax.experimental import pallas as pl
from jax.experimental.pallas import tpu as pltpu

B = 8
N = 1024
K = 16
C_IN = 64
C_OUT = 128
C_LIFT = 64
TILE = 256
Q = N // 4
BIG = 1e9
HUGE = 3e9
IDX_BIG = 1e9
HI = jax.lax.Precision.HIGHEST


def _ce(va, ia, vb, ib):
    pred = (va > vb) | ((va == vb) & (ia > ib))
    return (jnp.where(pred, vb, va), jnp.where(pred, ib, ia),
            jnp.where(pred, va, vb), jnp.where(pred, ia, ib))


def _xconv_kernel(q_ref, p_ref, pT_ref, feat_ref, W1_ref, b1_ref, W2_ref,
                  b2_ref, Wl_ref, bl_ref, Wf_ref, bf_ref, out_ref):
    q = q_ref[0]
    p_full = p_ref[0]
    pT = pT_ref[0]
    feat = feat_ref[0]

    p0 = pT[0:1, :]
    p1 = pT[1:2, :]
    p2 = pT[2:3, :]
    valid_p = (p0 != 0.0) | (p1 != 0.0) | (p2 != 0.0)
    valid_col = jnp.any(p_full != 0.0, axis=1, keepdims=True)
    valid_q = jnp.any(q != 0.0, axis=1, keepdims=True)

    d0 = q[:, 0:1] - p0
    d1 = q[:, 1:2] - p1
    d2 = q[:, 2:3] - p2
    pd = d0 * d0 + d1 * d1 + d2 * d2
    pd = jnp.where(valid_q & valid_p, pd, BIG)

    lifted = jax.nn.relu(jnp.dot(feat, Wl_ref[...],
                                 preferred_element_type=jnp.float32)
                         + bl_ref[...])
    lifted = jnp.where(valid_col, lifted, 0.0)

    payload = jnp.concatenate(
        [p_full, valid_col.astype(jnp.float32), lifted], axis=1)

    base = jax.lax.broadcasted_iota(jnp.int32, (TILE, Q), 1).astype(jnp.float32)
    v0, i0 = pd[:, 0:Q], base
    v1, i1 = pd[:, Q:2 * Q], base + float(Q)
    v2, i2 = pd[:, 2 * Q:3 * Q], base + float(2 * Q)
    v3, i3 = pd[:, 3 * Q:4 * Q], base + float(3 * Q)
    v0, i0, v1, i1 = _ce(v0, i0, v1, i1)
    v2, i2, v3, i3 = _ce(v2, i2, v3, i3)
    v0, i0, v2, i2 = _ce(v0, i0, v2, i2)
    v1, i1, v3, i3 = _ce(v1, i1, v3, i3)
    v1, i1, v2, i2 = _ce(v1, i1, v2, i2)

    iota_full = jax.lax.broadcasted_iota(
        jnp.int32, (TILE, N), 1).astype(jnp.float32)

    h_parts = []
    L_parts = []
    for _ in range(K):
        m = jnp.min(v0, axis=1, keepdims=True)
        cand = jnp.where(v0 <= m, i0, IDX_BIG)
        argi = jnp.min(cand, axis=1, keepdims=True)
        pool_oh = i0 == argi
        v0 = jnp.where(pool_oh, v1, v0)
        i0 = jnp.where(pool_oh, i1, i0)
        v1 = jnp.where(pool_oh, v2, v1)
        i1 = jnp.where(pool_oh, i2, i1)
        v2 = jnp.where(pool_oh, v3, v2)
        i2 = jnp.where(pool_oh, i3, i2)
        v3 = jnp.where(pool_oh, HUGE, v3)
        i3 = jnp.where(pool_oh, IDX_BIG, i3)

        ohf = jnp.where(iota_full == argi, 1.0, 0.0)
        g = jnp.dot(ohf, payload, preferred_element_type=jnp.float32)
        nbr_valid = g[:, 3:4] > 0.5
        rel = jnp.where(nbr_valid, g[:, 0:3] - q, 0.0)
        h = jax.nn.relu(jnp.dot(rel, W1_ref[...],
                                preferred_element_type=jnp.float32)
                        + b1_ref[...])
        h_parts.append(h)
        L_parts.append(g[:, 4:4 + C_LIFT])

    h_flat = jnp.concatenate(h_parts, axis=1)
    Xp = jnp.dot(h_flat, W2_ref[...],
                 preferred_element_type=jnp.float32) + b2_ref[...]

    lane_e = jax.lax.broadcasted_iota(jnp.int32, (K, N), 1)
    row_e = jax.lax.broadcasted_iota(jnp.int32, (K, N), 0)
    E = (lane_e // C_LIFT == row_e).astype(jnp.float32)

    T = None
    for j in range(K):
        Xrep = jnp.dot(Xp[:, j * K:(j + 1) * K], E,
                       preferred_element_type=jnp.float32)
        Ltile = pltpu.repeat(L_parts[j], K, axis=1)
        T = Xrep * Ltile if T is None else T + Xrep * Ltile

    final = jax.nn.relu(jnp.dot(T, Wf_ref[...],
                                preferred_element_type=jnp.float32)
                        + bf_ref[...])
    out_ref[0] = jnp.where(valid_q, final, 0.0)


@jax.jit
def kernel(points_xyz, features, W1, b1, W2, b2, Wl, bl, Wf, bf):
    pT = jnp.transpose(points_xyz, (0, 2, 1))
    perm = np.arange(K * K).reshape(K, K).T.reshape(-1)
    W2p = W2[:, perm]
    b2p = b2[perm]
    grid = (B, N // TILE)
    out = pl.pallas_call(
        _xconv_kernel,
        grid=grid,
        in_specs=[
            pl.BlockSpec((1, TILE, 3), lambda b, t: (b, t, 0)),
            pl.BlockSpec((1, N, 3), lambda b, t: (b, 0, 0)),
            pl.BlockSpec((1, 3, N), lambda b, t: (b, 0, 0)),
            pl.BlockSpec((1, N, C_IN), lambda b, t: (b, 0, 0)),
            pl.BlockSpec(W1.shape, lambda b, t: (0, 0)),
            pl.BlockSpec((1, K * 2), lambda b, t: (0, 0)),
            pl.BlockSpec(W2.shape, lambda b, t: (0, 0)),
            pl.BlockSpec((1, K * K), lambda b, t: (0, 0)),
            pl.BlockSpec(Wl.shape, lambda b, t: (0, 0)),
            pl.BlockSpec((1, C_LIFT), lambda b, t: (0, 0)),
            pl.BlockSpec(Wf.shape, lambda b, t: (0, 0)),
            pl.BlockSpec((1, C_OUT), lambda b, t: (0, 0)),
        ],
        out_specs=pl.BlockSpec((1, TILE, C_OUT), lambda b, t: (b, t, 0)),
        out_shape=jax.ShapeDtypeStruct((B, N, C_OUT), jnp.float32),
    )(points_xyz, points_xyz, pT, features, W1, b1.reshape(1, -1), W2p,
      b2p.reshape(1, -1), Wl, bl.reshape(1, -1), Wf, bf.reshape(1, -1))
    return out


# speedup vs baseline: 5.5038x; 1.0512x over previous
import jax
import jax.numpy as jnp
import numpy as np
from jax.experimental import pallas as pl
from jax.experimental.pallas import tpu as pltpu

B = 8
N = 1024
K = 16
C_IN = 64
C_OUT = 128
C_LIFT = 64
TILE = 256
Q = N // 4
BIG = 1e9
HUGE = 3e9
IDX_BIG = 1e9
HI = jax.lax.Precision.HIGHEST


def _ce(va, ia, vb, ib):
    pred = (va > vb) | ((va == vb) & (ia > ib))
    return (jnp.where(pred, vb, va), jnp.where(pred, ib, ia),
            jnp.where(pred, va, vb), jnp.where(pred, ia, ib))


def _xconv_kernel(q_ref, p_ref, pT_ref, feat_ref, W1_ref, b1_ref, W2_ref,
                  b2_ref, Wl_ref, bl_ref, Wf_ref, bf_ref, out_ref,
                  payload_ref):
    q = q_ref[0]
    p_full = p_ref[0]
    pT = pT_ref[0]
    feat = feat_ref[0]

    p0 = pT[0:1, :]
    p1 = pT[1:2, :]
    p2 = pT[2:3, :]
    valid_p = (p0 != 0.0) | (p1 != 0.0) | (p2 != 0.0)
    valid_col = jnp.any(p_full != 0.0, axis=1, keepdims=True)
    valid_q = jnp.any(q != 0.0, axis=1, keepdims=True)

    d0 = q[:, 0:1] - p0
    d1 = q[:, 1:2] - p1
    d2 = q[:, 2:3] - p2
    pd = d0 * d0 + d1 * d1 + d2 * d2
    pd = jnp.where(valid_q & valid_p, pd, BIG)

    @pl.when(pl.program_id(1) == 0)
    def _build_payload():
        lifted = jax.nn.relu(jnp.dot(feat, Wl_ref[...],
                                     preferred_element_type=jnp.float32)
                             + bl_ref[...])
        lifted = jnp.where(valid_col, lifted, 0.0)
        payload_ref[...] = jnp.concatenate(
            [p_full, valid_col.astype(jnp.float32), lifted], axis=1)

    payload = payload_ref[...]

    base = jax.lax.broadcasted_iota(jnp.int32, (TILE, Q), 1).astype(jnp.float32)
    v0, i0 = pd[:, 0:Q], base
    v1, i1 = pd[:, Q:2 * Q], base + float(Q)
    v2, i2 = pd[:, 2 * Q:3 * Q], base + float(2 * Q)
    v3, i3 = pd[:, 3 * Q:4 * Q], base + float(3 * Q)
    v0, i0, v1, i1 = _ce(v0, i0, v1, i1)
    v2, i2, v3, i3 = _ce(v2, i2, v3, i3)
    v0, i0, v2, i2 = _ce(v0, i0, v2, i2)
    v1, i1, v3, i3 = _ce(v1, i1, v3, i3)
    v1, i1, v2, i2 = _ce(v1, i1, v2, i2)

    iota_full = jax.lax.broadcasted_iota(
        jnp.int32, (TILE, N), 1).astype(jnp.float32)

    h_parts = []
    L_parts = []
    for _ in range(K):
        m = jnp.min(v0, axis=1, keepdims=True)
        cand = jnp.where(v0 <= m, i0, IDX_BIG)
        argi = jnp.min(cand, axis=1, keepdims=True)
        pool_oh = i0 == argi
        v0 = jnp.where(pool_oh, v1, v0)
        i0 = jnp.where(pool_oh, i1, i0)
        v1 = jnp.where(pool_oh, v2, v1)
        i1 = jnp.where(pool_oh, i2, i1)
        v2 = jnp.where(pool_oh, v3, v2)
        i2 = jnp.where(pool_oh, i3, i2)
        v3 = jnp.where(pool_oh, HUGE, v3)
        i3 = jnp.where(pool_oh, IDX_BIG, i3)

        ohf = jnp.where(iota_full == argi, 1.0, 0.0)
        g = jnp.dot(ohf, payload, preferred_element_type=jnp.float32)
        nbr_valid = g[:, 3:4] > 0.5
        rel = jnp.where(nbr_valid, g[:, 0:3] - q, 0.0)
        h = jax.nn.relu(jnp.dot(rel, W1_ref[...],
                                preferred_element_type=jnp.float32)
                        + b1_ref[...])
        h_parts.append(h)
        L_parts.append(g[:, 4:4 + C_LIFT])

    h_flat = jnp.concatenate(h_parts, axis=1)
    Xp = jnp.dot(h_flat, W2_ref[...],
                 preferred_element_type=jnp.float32) + b2_ref[...]

    lane_e = jax.lax.broadcasted_iota(jnp.int32, (K, N), 1)
    row_e = jax.lax.broadcasted_iota(jnp.int32, (K, N), 0)
    E = (lane_e // C_LIFT == row_e).astype(jnp.float32)

    T = None
    for j in range(K):
        Xrep = jnp.dot(Xp[:, j * K:(j + 1) * K], E,
                       preferred_element_type=jnp.float32)
        Ltile = pltpu.repeat(L_parts[j], K, axis=1)
        T = Xrep * Ltile if T is None else T + Xrep * Ltile

    final = jax.nn.relu(jnp.dot(T, Wf_ref[...],
                                preferred_element_type=jnp.float32)
                        + bf_ref[...])
    out_ref[0] = jnp.where(valid_q, final, 0.0)


@jax.jit
def kernel(points_xyz, features, W1, b1, W2, b2, Wl, bl, Wf, bf):
    pT = jnp.transpose(points_xyz, (0, 2, 1))
    perm = np.arange(K * K).reshape(K, K).T.reshape(-1)
    W2p = W2[:, perm]
    b2p = b2[perm]
    grid = (B, N // TILE)
    out = pl.pallas_call(
        _xconv_kernel,
        grid=grid,
        in_specs=[
            pl.BlockSpec((1, TILE, 3), lambda b, t: (b, t, 0)),
            pl.BlockSpec((1, N, 3), lambda b, t: (b, 0, 0)),
            pl.BlockSpec((1, 3, N), lambda b, t: (b, 0, 0)),
            pl.BlockSpec((1, N, C_IN), lambda b, t: (b, 0, 0)),
            pl.BlockSpec(W1.shape, lambda b, t: (0, 0)),
            pl.BlockSpec((1, K * 2), lambda b, t: (0, 0)),
            pl.BlockSpec(W2.shape, lambda b, t: (0, 0)),
            pl.BlockSpec((1, K * K), lambda b, t: (0, 0)),
            pl.BlockSpec(Wl.shape, lambda b, t: (0, 0)),
            pl.BlockSpec((1, C_LIFT), lambda b, t: (0, 0)),
            pl.BlockSpec(Wf.shape, lambda b, t: (0, 0)),
            pl.BlockSpec((1, C_OUT), lambda b, t: (0, 0)),
        ],
        out_specs=pl.BlockSpec((1, TILE, C_OUT), lambda b, t: (b, t, 0)),
        out_shape=jax.ShapeDtypeStruct((B, N, C_OUT), jnp.float32),
        scratch_shapes=[pltpu.VMEM((N, 4 + C_LIFT), jnp.float32)],
    )(points_xyz, points_xyz, pT, features, W1, b1.reshape(1, -1), W2p,
      b2p.reshape(1, -1), Wl, bl.reshape(1, -1), Wf, bf.reshape(1, -1))
    return out


# speedup vs baseline: 6.0814x; 1.1050x over previous
import jax
import jax.numpy as jnp
import numpy as np
from jax.experimental import pallas as pl
from jax.experimental.pallas import tpu as pltpu

B = 8
N = 1024
K = 16
C_IN = 64
C_OUT = 128
C_LIFT = 64
TILE = 512
Q = N // 4
BIG = 1e9
HUGE = 3e9
IDX_BIG = 1e9
HI = jax.lax.Precision.HIGHEST


def _ce(va, ia, vb, ib):
    pred = (va > vb) | ((va == vb) & (ia > ib))
    return (jnp.where(pred, vb, va), jnp.where(pred, ib, ia),
            jnp.where(pred, va, vb), jnp.where(pred, ia, ib))


def _xconv_kernel(q_ref, p_ref, pT_ref, feat_ref, W1_ref, b1_ref, W2_ref,
                  b2_ref, Wl_ref, bl_ref, Wf_ref, bf_ref, out_ref,
                  payload_ref):
    q = q_ref[0]
    p_full = p_ref[0]
    pT = pT_ref[0]
    feat = feat_ref[0]

    p0 = pT[0:1, :]
    p1 = pT[1:2, :]
    p2 = pT[2:3, :]
    valid_p = (p0 != 0.0) | (p1 != 0.0) | (p2 != 0.0)
    valid_col = jnp.any(p_full != 0.0, axis=1, keepdims=True)
    valid_q = jnp.any(q != 0.0, axis=1, keepdims=True)

    d0 = q[:, 0:1] - p0
    d1 = q[:, 1:2] - p1
    d2 = q[:, 2:3] - p2
    pd = d0 * d0 + d1 * d1 + d2 * d2
    pd = jnp.where(valid_q & valid_p, pd, BIG)

    @pl.when(pl.program_id(1) == 0)
    def _build_payload():
        lifted = jax.nn.relu(jnp.dot(feat, Wl_ref[...],
                                     preferred_element_type=jnp.float32)
                             + bl_ref[...])
        lifted = jnp.where(valid_col, lifted, 0.0)
        payload_ref[...] = jnp.concatenate(
            [p_full, valid_col.astype(jnp.float32), lifted], axis=1)

    payload = payload_ref[...]

    base = jax.lax.broadcasted_iota(jnp.int32, (TILE, Q), 1).astype(jnp.float32)
    v0, i0 = pd[:, 0:Q], base
    v1, i1 = pd[:, Q:2 * Q], base + float(Q)
    v2, i2 = pd[:, 2 * Q:3 * Q], base + float(2 * Q)
    v3, i3 = pd[:, 3 * Q:4 * Q], base + float(3 * Q)
    v0, i0, v1, i1 = _ce(v0, i0, v1, i1)
    v2, i2, v3, i3 = _ce(v2, i2, v3, i3)
    v0, i0, v2, i2 = _ce(v0, i0, v2, i2)
    v1, i1, v3, i3 = _ce(v1, i1, v3, i3)
    v1, i1, v2, i2 = _ce(v1, i1, v2, i2)

    iota_full = jax.lax.broadcasted_iota(
        jnp.int32, (TILE, N), 1).astype(jnp.float32)

    h_parts = []
    L_parts = []
    for _ in range(K):
        m = jnp.min(v0, axis=1, keepdims=True)
        cand = jnp.where(v0 <= m, i0, IDX_BIG)
        argi = jnp.min(cand, axis=1, keepdims=True)
        pool_oh = i0 == argi
        v0 = jnp.where(pool_oh, v1, v0)
        i0 = jnp.where(pool_oh, i1, i0)
        v1 = jnp.where(pool_oh, v2, v1)
        i1 = jnp.where(pool_oh, i2, i1)
        v2 = jnp.where(pool_oh, v3, v2)
        i2 = jnp.where(pool_oh, i3, i2)
        v3 = jnp.where(pool_oh, HUGE, v3)
        i3 = jnp.where(pool_oh, IDX_BIG, i3)

        ohf = jnp.where(iota_full == argi, 1.0, 0.0)
        g = jnp.dot(ohf, payload, preferred_element_type=jnp.float32)
        nbr_valid = g[:, 3:4] > 0.5
        rel = jnp.where(nbr_valid, g[:, 0:3] - q, 0.0)
        h = jax.nn.relu(jnp.dot(rel, W1_ref[...],
                                preferred_element_type=jnp.float32)
                        + b1_ref[...])
        h_parts.append(h)
        L_parts.append(g[:, 4:4 + C_LIFT])

    h_flat = jnp.concatenate(h_parts, axis=1)
    Xp = jnp.dot(h_flat, W2_ref[...],
                 preferred_element_type=jnp.float32) + b2_ref[...]

    lane_e = jax.lax.broadcasted_iota(jnp.int32, (K, N), 1)
    row_e = jax.lax.broadcasted_iota(jnp.int32, (K, N), 0)
    E = (lane_e // C_LIFT == row_e).astype(jnp.float32)

    T = None
    for j in range(K):
        Xrep = jnp.dot(Xp[:, j * K:(j + 1) * K], E,
                       preferred_element_type=jnp.float32)
        Ltile = pltpu.repeat(L_parts[j], K, axis=1)
        T = Xrep * Ltile if T is None else T + Xrep * Ltile

    final = jax.nn.relu(jnp.dot(T, Wf_ref[...],
                                preferred_element_type=jnp.float32)
                        + bf_ref[...])
    out_ref[0] = jnp.where(valid_q, final, 0.0)


@jax.jit
def kernel(points_xyz, features, W1, b1, W2, b2, Wl, bl, Wf, bf):
    pT = jnp.transpose(points_xyz, (0, 2, 1))
    perm = np.arange(K * K).reshape(K, K).T.reshape(-1)
    W2p = W2[:, perm]
    b2p = b2[perm]
    grid = (B, N // TILE)
    out = pl.pallas_call(
        _xconv_kernel,
        grid=grid,
        in_specs=[
            pl.BlockSpec((1, TILE, 3), lambda b, t: (b, t, 0)),
            pl.BlockSpec((1, N, 3), lambda b, t: (b, 0, 0)),
            pl.BlockSpec((1, 3, N), lambda b, t: (b, 0, 0)),
            pl.BlockSpec((1, N, C_IN), lambda b, t: (b, 0, 0)),
            pl.BlockSpec(W1.shape, lambda b, t: (0, 0)),
            pl.BlockSpec((1, K * 2), lambda b, t: (0, 0)),
            pl.BlockSpec(W2.shape, lambda b, t: (0, 0)),
            pl.BlockSpec((1, K * K), lambda b, t: (0, 0)),
            pl.BlockSpec(Wl.shape, lambda b, t: (0, 0)),
            pl.BlockSpec((1, C_LIFT), lambda b, t: (0, 0)),
            pl.BlockSpec(Wf.shape, lambda b, t: (0, 0)),
            pl.BlockSpec((1, C_OUT), lambda b, t: (0, 0)),
        ],
        out_specs=pl.BlockSpec((1, TILE, C_OUT), lambda b, t: (b, t, 0)),
        out_shape=jax.ShapeDtypeStruct((B, N, C_OUT), jnp.float32),
        scratch_shapes=[pltpu.VMEM((N, 4 + C_LIFT), jnp.float32)],
    )(points_xyz, points_xyz, pT, features, W1, b1.reshape(1, -1), W2p,
      b2p.reshape(1, -1), Wl, bl.reshape(1, -1), Wf, bf.reshape(1, -1))
    return out
